# Initial kernel scaffold; baseline (speedup 1.0000x reference)
import functools

import jax
import jax.numpy as jnp
from jax import lax
from jax.experimental import pallas as pl
from jax.experimental.pallas import tpu as pltpu
from jax.experimental.pallas import tpu_sc as plsc

N = 10000
E = 160000
IN_DIM = 256
OUT_DIM = 64
HEADS = 4
HD = HEADS * OUT_DIM
HALF = HD // 2

NC = 2
NS = 16

CH2 = E // (NC * NS)
SUB2 = 1000
BLK2 = 40

CH3 = E // NS
BLK3 = 80
NB3 = CH3 // BLK3

ROWS_PER_TILE = N // NS
ZROWS = 125

MB = 1000
GRID = N // MB


def _k1_body(x_ref, wt_ref, b_ref, asrc_ref, adst_ref,
             h01_ref, h23_ref, ss_ref, sd_ref):
    xb = x_ref[...]
    h = jnp.dot(xb, wt_ref[...], preferred_element_type=jnp.float32)
    h = h + b_ref[...]
    ts = h * asrc_ref[...]
    td = h * adst_ref[...]
    ss_cols = [jnp.sum(ts[:, i * OUT_DIM:(i + 1) * OUT_DIM], axis=1,
                       keepdims=True) for i in range(HEADS)]
    sd_cols = [jnp.sum(td[:, i * OUT_DIM:(i + 1) * OUT_DIM], axis=1,
                       keepdims=True) for i in range(HEADS)]
    ss_ref[...] = jnp.concatenate(ss_cols, axis=1)
    sd_ref[...] = jnp.concatenate(sd_cols, axis=1)
    h01_ref[...] = h[:, :HALF]
    h23_ref[...] = h[:, HALF:]


def _k1(x, wt, b2, asrc2, adst2):
    return pl.pallas_call(
        _k1_body,
        grid=(GRID,),
        in_specs=[
            pl.BlockSpec((MB, IN_DIM), lambda i: (i, 0)),
            pl.BlockSpec((IN_DIM, HD), lambda i: (0, 0)),
            pl.BlockSpec((1, HD), lambda i: (0, 0)),
            pl.BlockSpec((1, HD), lambda i: (0, 0)),
            pl.BlockSpec((1, HD), lambda i: (0, 0)),
        ],
        out_specs=[
            pl.BlockSpec((MB, HALF), lambda i: (i, 0)),
            pl.BlockSpec((MB, HALF), lambda i: (i, 0)),
            pl.BlockSpec((MB, HEADS), lambda i: (i, 0)),
            pl.BlockSpec((MB, HEADS), lambda i: (i, 0)),
        ],
        out_shape=[
            jax.ShapeDtypeStruct((N, HALF), jnp.float32),
            jax.ShapeDtypeStruct((N, HALF), jnp.float32),
            jax.ShapeDtypeStruct((N, HEADS), jnp.float32),
            jax.ShapeDtypeStruct((N, HEADS), jnp.float32),
        ],
    )(x, wt, b2, asrc2, adst2)


def _k2_body(ss_hbm, sd_hbm, ei_hbm, attn_hbm, dpart_hbm,
             ssv, sdv, sidx, didx, didx2, attnb, expb, zb, denom_sh):
    c = lax.axis_index("c")
    s = lax.axis_index("s")
    tid = c * NS + s
    base = tid * CH2

    pltpu.sync_copy(ss_hbm, ssv)
    pltpu.sync_copy(sd_hbm, sdv)

    def _z(i, _):
        zb[pl.ds(i * 16, 16)] = jnp.zeros((16,), jnp.float32)
        return 0
    lax.fori_loop(0, (1250 * HEADS) // 16, _z, 0)

    @pl.when(s < 8)
    def _():
        pltpu.sync_copy(zb.reshape(1250, HEADS),
                        denom_sh.at[pl.ds(s * 1250, 1250), :])

    plsc.subcore_barrier()

    iota = lax.broadcasted_iota(jnp.int32, (16,), 0)
    rep4 = lax.shift_right_logical(iota, 2)
    mod4 = lax.bitwise_and(iota, 3)

    for u in range(CH2 // SUB2):
        sub = base + u * SUB2
        pltpu.sync_copy(ei_hbm.at[0, pl.ds(sub, SUB2)], sidx)
        pltpu.sync_copy(ei_hbm.at[1, pl.ds(sub, SUB2)], didx)
        for j in range(SUB2 // BLK2):
            pltpu.sync_copy(ei_hbm.at[1, pl.ds(sub + j * BLK2, BLK2)],
                            didx2.at[j])

        def _grp(g, _):
            row = g * 4 + rep4
            sv = plsc.load_gather(sidx, [row])
            dv = plsc.load_gather(didx, [row])
            a = plsc.load_gather(ssv, [sv, mod4])
            b = plsc.load_gather(sdv, [dv, mod4])
            t = a + b
            t = jnp.where(t >= 0.0, t, 0.2 * t)
            attnb[pl.ds(g * 16, 16)] = t
            plsc.store_scatter(expb, [row, mod4], jnp.exp(t))
            return 0
        lax.fori_loop(0, (SUB2 * HEADS) // 16, _grp, 0)

        pltpu.sync_copy(attnb, attn_hbm.at[pl.ds(sub * HEADS, SUB2 * HEADS)])
        for j in range(SUB2 // BLK2):
            pltpu.sync_copy(expb.at[pl.ds(j * BLK2, BLK2), :],
                            denom_sh.at[didx2.at[j]], add=True)

    plsc.subcore_barrier()

    @pl.when(s < 8)
    def _():
        pltpu.sync_copy(denom_sh.at[pl.ds(s * 1250, 1250), :],
                        dpart_hbm.at[c, pl.ds(s * 1250, 1250), :])


def _k2(ss, sd, ei):
    mesh = plsc.VectorSubcoreMesh(core_axis_name="c", subcore_axis_name="s",
                                  num_cores=NC, num_subcores=NS)
    f = pl.kernel(
        _k2_body,
        out_type=(
            jax.ShapeDtypeStruct((E * HEADS,), jnp.float32),
            jax.ShapeDtypeStruct((NC, N, HEADS), jnp.float32),
        ),
        mesh=mesh,
        scratch_types=[
            pltpu.VMEM((N, HEADS), jnp.float32),
            pltpu.VMEM((N, HEADS), jnp.float32),
            pltpu.VMEM((SUB2,), jnp.int32),
            pltpu.VMEM((SUB2,), jnp.int32),
            pltpu.VMEM((SUB2 // BLK2, BLK2), jnp.int32),
            pltpu.VMEM((SUB2 * HEADS,), jnp.float32),
            pltpu.VMEM((SUB2, HEADS), jnp.float32),
            pltpu.VMEM((1250 * HEADS,), jnp.float32),
            pltpu.VMEM_SHARED((N, HEADS), jnp.float32),
        ],
    )
    return f(ss, sd, ei)


def _k2b_body(dp_ref, dn_ref):
    dn_ref[...] = dp_ref[0] + dp_ref[1] + 1e-08


def _k2b(dparts):
    return pl.pallas_call(
        _k2b_body,
        out_shape=jax.ShapeDtypeStruct((N, HEADS), jnp.float32),
    )(dparts)


def _k3_body(h01_hbm, h23_hbm, attn_hbm, dn_hbm, ei_hbm, out_hbm,
             dnv, sidx, didx, didx2, attnb, alphab, hrows, wrows, zb, acc_sh):
    c = lax.axis_index("c")
    s = lax.axis_index("s")
    base = s * CH3

    pltpu.sync_copy(dn_hbm, dnv)
    pltpu.sync_copy(ei_hbm.at[0, pl.ds(base, CH3)], sidx)
    pltpu.sync_copy(ei_hbm.at[1, pl.ds(base, CH3)], didx)
    for j in range(NB3):
        pltpu.sync_copy(ei_hbm.at[1, pl.ds(base + j * BLK3, BLK3)],
                        didx2.at[j])

    def _z(i, _):
        zb[pl.ds(i * 16, 16)] = jnp.zeros((16,), jnp.float32)
        return 0
    lax.fori_loop(0, (ZROWS * HALF) // 16, _z, 0)
    for k in range(ROWS_PER_TILE // ZROWS):
        pltpu.sync_copy(zb.reshape(ZROWS, HALF),
                        acc_sh.at[pl.ds(s * ROWS_PER_TILE + k * ZROWS, ZROWS), :])

    plsc.subcore_barrier()

    iota = lax.broadcasted_iota(jnp.int32, (16,), 0)
    rep4 = lax.shift_right_logical(iota, 2)
    mod4 = lax.bitwise_and(iota, 3)

    def _blk(bidx, _):
        idxs = sidx.at[pl.ds(bidx * BLK3, BLK3)]

        @pl.when(c == 0)
        def _():
            pltpu.sync_copy(h01_hbm.at[idxs], hrows)

        @pl.when(c == 1)
        def _():
            pltpu.sync_copy(h23_hbm.at[idxs], hrows)

        pltpu.sync_copy(
            attn_hbm.at[pl.ds((base + bidx * BLK3) * HEADS, BLK3 * HEADS)],
            attnb)

        def _grp(g, _):
            row = bidx * BLK3 + g * 4 + rep4
            dv = plsc.load_gather(didx, [row])
            dn = plsc.load_gather(dnv, [dv, mod4])
            at = attnb[pl.ds(g * 16, 16)]
            alphab[pl.ds(g * 16, 16)] = at / dn
            return 0
        lax.fori_loop(0, (BLK3 * HEADS) // 16, _grp, 0)

        def _we(e, _):
            for hh in range(2):
                ai = jnp.full((16,), e * HEADS + 2 * c + hh, jnp.int32)
                av = plsc.load_gather(alphab, [ai])
                for q in range(OUT_DIM // 16):
                    col = hh * OUT_DIM + q * 16
                    wrows[e, pl.ds(col, 16)] = hrows[e, pl.ds(col, 16)] * av
            return 0
        lax.fori_loop(0, BLK3, _we, 0)

        pltpu.sync_copy(wrows, acc_sh.at[didx2.at[bidx]], add=True)
        return 0
    lax.fori_loop(0, NB3, _blk, 0)

    plsc.subcore_barrier()

    for k in range(ROWS_PER_TILE // ZROWS):
        r0 = s * ROWS_PER_TILE + k * ZROWS
        pltpu.sync_copy(acc_sh.at[pl.ds(r0, ZROWS), :],
                        out_hbm.at[c, pl.ds(r0, ZROWS), :])


def _k3(h01, h23, attn, dn, ei):
    mesh = plsc.VectorSubcoreMesh(core_axis_name="c", subcore_axis_name="s",
                                  num_cores=NC, num_subcores=NS)
    f = pl.kernel(
        _k3_body,
        out_type=jax.ShapeDtypeStruct((NC, N, HALF), jnp.float32),
        mesh=mesh,
        scratch_types=[
            pltpu.VMEM((N, HEADS), jnp.float32),
            pltpu.VMEM((CH3,), jnp.int32),
            pltpu.VMEM((CH3,), jnp.int32),
            pltpu.VMEM((NB3, BLK3), jnp.int32),
            pltpu.VMEM((BLK3 * HEADS,), jnp.float32),
            pltpu.VMEM((BLK3 * HEADS,), jnp.float32),
            pltpu.VMEM((BLK3, HALF), jnp.float32),
            pltpu.VMEM((BLK3, HALF), jnp.float32),
            pltpu.VMEM((ZROWS * HALF,), jnp.float32),
            pltpu.VMEM_SHARED((N, HALF), jnp.float32),
        ],
    )
    return f(h01, h23, attn, dn, ei)


def _k4a_body(acc_ref, sums_ref):
    i = pl.program_id(0)
    r = jnp.concatenate([acc_ref[0], acc_ref[1]], axis=-1)
    r = jnp.maximum(r, 0.0)
    s1 = jnp.sum(r, axis=0, keepdims=True)
    s2 = jnp.sum(r * r, axis=0, keepdims=True)
    cur = jnp.concatenate([s1, s2], axis=0)

    @pl.when(i == 0)
    def _():
        sums_ref[...] = cur

    @pl.when(i > 0)
    def _():
        sums_ref[...] = sums_ref[...] + cur


def _k4a(acc):
    return pl.pallas_call(
        _k4a_body,
        grid=(GRID,),
        in_specs=[pl.BlockSpec((NC, MB, HALF), lambda i: (0, i, 0))],
        out_specs=pl.BlockSpec((2, HD), lambda i: (0, 0)),
        out_shape=jax.ShapeDtypeStruct((2, HD), jnp.float32),
    )(acc)


def _k4b_body(acc_ref, sums_ref, g_ref, b_ref, out_ref):
    r = jnp.concatenate([acc_ref[0], acc_ref[1]], axis=-1)
    r = jnp.maximum(r, 0.0)
    mean = sums_ref[0:1, :] * (1.0 / N)
    var = sums_ref[1:2, :] * (1.0 / N) - mean * mean
    inv = lax.rsqrt(var + 1e-05)
    out_ref[...] = (r - mean) * inv * g_ref[...] + b_ref[...]


def _k4b(acc, sums, gamma2, beta2):
    return pl.pallas_call(
        _k4b_body,
        grid=(GRID,),
        in_specs=[
            pl.BlockSpec((NC, MB, HALF), lambda i: (0, i, 0)),
            pl.BlockSpec((2, HD), lambda i: (0, 0)),
            pl.BlockSpec((1, HD), lambda i: (0, 0)),
            pl.BlockSpec((1, HD), lambda i: (0, 0)),
        ],
        out_specs=pl.BlockSpec((MB, HD), lambda i: (i, 0)),
        out_shape=jax.ShapeDtypeStruct((N, HD), jnp.float32),
    )(acc, sums, gamma2, beta2)


@jax.jit
def kernel(x, edge_index, W_w, W_b, a_src_w, a_dst_w, bn_gamma, bn_beta):
    wt = W_w.T
    b2 = W_b.reshape(1, HD)
    asrc2 = jnp.tile(a_src_w[0], HEADS).reshape(1, HD)
    adst2 = jnp.tile(a_dst_w[0], HEADS).reshape(1, HD)

    h01, h23, ss, sd = _k1(x, wt, b2, asrc2, adst2)
    attn, dparts = _k2(ss, sd, edge_index)
    dn = _k2b(dparts)
    acc = _k3(h01, h23, attn, dn, edge_index)
    sums = _k4a(acc)
    out = _k4b(acc, sums, bn_gamma.reshape(1, HD), bn_beta.reshape(1, HD))
    return out


# speedup vs baseline: 54.3925x; 54.3925x over previous
import jax
import jax.numpy as jnp
from jax import lax
from jax.experimental import pallas as pl
from jax.experimental.pallas import tpu as pltpu
from jax.experimental.pallas import tpu_sc as plsc

N = 10000
E = 160000
IN_DIM = 256
OUT_DIM = 64
HEADS = 4
HD = HEADS * OUT_DIM
HALF = HD // 2

NC = 2
NS = 16

BLK2 = 128
NBLK2 = E // BLK2
NB2_BASE = NBLK2 // (NC * NS)

BLK3 = 128
NB3_BASE = NBLK2 // NS

WTILES = 10
WROWS = N // WTILES
ZROWS = 40

NP = 10240
DCH = 2048
NDCH = NP // DCH

MB = 1000
GRID = N // MB


def _k1_body(x_ref, wt_ref, b_ref, asrc_ref, adst_ref,
             h01_ref, h23_ref, ss_ref, sd_ref):
    xb = x_ref[...]
    h = jnp.dot(xb, wt_ref[...], preferred_element_type=jnp.float32)
    h = h + b_ref[...]
    ts = h * asrc_ref[...]
    td = h * adst_ref[...]
    ss_ref[...] = jnp.concatenate(
        [jnp.sum(ts[:, i * OUT_DIM:(i + 1) * OUT_DIM], axis=1, keepdims=True)
         for i in range(HEADS)], axis=1)
    sd_ref[...] = jnp.concatenate(
        [jnp.sum(td[:, i * OUT_DIM:(i + 1) * OUT_DIM], axis=1, keepdims=True)
         for i in range(HEADS)], axis=1)
    h01_ref[...] = h[:, :HALF]
    h23_ref[...] = h[:, HALF:]


def _k1(x, wt, b2, asrc2, adst2):
    return pl.pallas_call(
        _k1_body,
        grid=(GRID,),
        in_specs=[
            pl.BlockSpec((MB, IN_DIM), lambda i: (i, 0)),
            pl.BlockSpec((IN_DIM, HD), lambda i: (0, 0)),
            pl.BlockSpec((1, HD), lambda i: (0, 0)),
            pl.BlockSpec((1, HD), lambda i: (0, 0)),
            pl.BlockSpec((1, HD), lambda i: (0, 0)),
        ],
        out_specs=[
            pl.BlockSpec((MB, HALF), lambda i: (i, 0)),
            pl.BlockSpec((MB, HALF), lambda i: (i, 0)),
            pl.BlockSpec((MB, HEADS), lambda i: (i, 0)),
            pl.BlockSpec((MB, HEADS), lambda i: (i, 0)),
        ],
        out_shape=[
            jax.ShapeDtypeStruct((N, HALF), jnp.float32),
            jax.ShapeDtypeStruct((N, HALF), jnp.float32),
            jax.ShapeDtypeStruct((N, HEADS), jnp.float32),
            jax.ShapeDtypeStruct((N, HEADS), jnp.float32),
        ],
    )(x, wt, b2, asrc2, adst2)


def _k2_body(ss0, ss1, ss2, ss3, sd0, sd1, sd2, sd3, esrc, edst,
             a_all, dparts,
             sidx2, didx2, sg2, dg2, attnb, expb2, zb,
             den0, den1, den2, den3, semg, semw):
    c = lax.axis_index("c")
    s = lax.axis_index("s")
    tid = c * NS + s

    ss_t = [ss0, ss1, ss2, ss3]
    sd_t = [sd0, sd1, sd2, sd3]
    den_t = [den0, den1, den2, den3]

    zval = jnp.zeros((16,), jnp.float32)

    def _z(i, _):
        zb[pl.ds(i * 16, 16)] = zval
        return 0
    lax.fori_loop(0, DCH // 16, _z, 0)
    for h in range(HEADS):
        for k in range(NDCH):
            job = h * NDCH + k

            @pl.when(s == job % NS)
            def _(h=h, k=k):
                pltpu.sync_copy(zb, den_t[h].at[pl.ds(k * DCH, DCH)])

    plsc.subcore_barrier()

    nb = NB2_BASE + jnp.where(tid < 2, 1, 0)
    jbase = tid * NB2_BASE + jnp.minimum(tid, 2)

    def _prefetch(b, p):
        j = jbase + b
        pltpu.sync_copy(esrc.at[j, 0, :], sidx2.at[p])
        pltpu.sync_copy(edst.at[j, 0, :], didx2.at[p])
        for h in range(HEADS):
            pltpu.async_copy(ss_t[h].at[sidx2.at[p]], sg2.at[p, h],
                             semg.at[p])
            pltpu.async_copy(sd_t[h].at[didx2.at[p]], dg2.at[p, h],
                             semg.at[p])

    _prefetch(0, 0)

    def _blk(b, _):
        p = lax.bitwise_and(b, 1)
        q = 1 - p

        @pl.when(b >= 1)
        def _():
            for h in range(HEADS):
                pltpu.make_async_copy(ss0.at[pl.ds(0, BLK2)],
                                      expb2.at[q, h], semw.at[q]).wait()

        @pl.when(b + 1 < nb)
        def _():
            _prefetch(b + 1, q)

        for h in range(HEADS):
            pltpu.make_async_copy(ss0.at[pl.ds(0, BLK2)], sg2.at[p, h],
                                  semg.at[p]).wait()
            pltpu.make_async_copy(ss0.at[pl.ds(0, BLK2)], dg2.at[p, h],
                                  semg.at[p]).wait()

        for h in range(HEADS):
            for g in range(BLK2 // 16):
                v = (sg2[p, h, pl.ds(g * 16, 16)]
                     + dg2[p, h, pl.ds(g * 16, 16)])
                t = jnp.where(v >= 0.0, v, 0.2 * v)
                attnb[h, pl.ds(g * 16, 16)] = t
                expb2[p, h, pl.ds(g * 16, 16)] = jnp.exp(t)

        j = jbase + b
        for h in range(HEADS):
            pltpu.async_copy(expb2.at[p, h], den_t[h].at[didx2.at[p]],
                             semw.at[p], add=True)
        pltpu.sync_copy(attnb, a_all.at[j, :, :])
        return 0
    lax.fori_loop(0, nb, _blk, 0)

    lastp = lax.bitwise_and(nb - 1, 1)
    for h in range(HEADS):
        pltpu.make_async_copy(ss0.at[pl.ds(0, BLK2)],
                              expb2.at[lastp, h], semw.at[lastp]).wait()

    plsc.subcore_barrier()

    for h in range(HEADS):
        for k in range(NDCH):
            job = h * NDCH + k

            @pl.when(s == job % NS)
            def _(h=h, k=k):
                pltpu.sync_copy(den_t[h].at[pl.ds(k * DCH, DCH)],
                                dparts.at[c, h, 0, pl.ds(k * DCH, DCH)])


def _k2(ss, sd, esrc, edst):
    mesh = plsc.VectorSubcoreMesh(core_axis_name="c", subcore_axis_name="s",
                                  num_cores=NC, num_subcores=NS)
    f = pl.kernel(
        _k2_body,
        out_type=(
            jax.ShapeDtypeStruct((NBLK2, HEADS, BLK2), jnp.float32),
            jax.ShapeDtypeStruct((NC, HEADS, 1, NP), jnp.float32),
        ),
        mesh=mesh,
        scratch_types=[
            pltpu.VMEM((2, BLK2), jnp.int32),
            pltpu.VMEM((2, BLK2), jnp.int32),
            pltpu.VMEM((2, HEADS, BLK2), jnp.float32),
            pltpu.VMEM((2, HEADS, BLK2), jnp.float32),
            pltpu.VMEM((HEADS, BLK2), jnp.float32),
            pltpu.VMEM((2, HEADS, BLK2), jnp.float32),
            pltpu.VMEM((DCH,), jnp.float32),
            pltpu.VMEM_SHARED((NP,), jnp.float32),
            pltpu.VMEM_SHARED((NP,), jnp.float32),
            pltpu.VMEM_SHARED((NP,), jnp.float32),
            pltpu.VMEM_SHARED((NP,), jnp.float32),
            pltpu.SemaphoreType.DMA((2,)),
            pltpu.SemaphoreType.DMA((2,)),
        ],
    )
    return f(ss[0], ss[1], ss[2], ss[3], sd[0], sd[1], sd[2], sd[3],
             esrc, edst)


def _k2b_body(dp_ref, dn_ref):
    dn_ref[...] = dp_ref[0, :, 0, :] + dp_ref[1, :, 0, :] + 1e-08


def _k2b(dparts):
    return pl.pallas_call(
        _k2b_body,
        out_shape=jax.ShapeDtypeStruct((HEADS, NP), jnp.float32),
    )(dparts)


def _k3_body(h01, h23, a_all, d0, d1, d2, d3, esrc, edst,
             out_hbm,
             sidx2, didx3, hrows3, attnb2, dnb2, acc_sh, semg, sems):
    c = lax.axis_index("c")
    s = lax.axis_index("s")
    nb = NB3_BASE + jnp.where(s < 2, 1, 0)
    jbase = s * NB3_BASE + jnp.minimum(s, 2)

    zval = jnp.zeros((16,), jnp.float32)

    def _z(i, _):
        r = lax.shift_right_logical(i, 3)
        q = lax.bitwise_and(i, 7)
        hrows3[0, r, pl.ds(q * 16, 16)] = zval
        return 0
    lax.fori_loop(0, ZROWS * (HALF // 16), _z, 0)

    @pl.when(s < WTILES)
    def _():
        for k in range(WROWS // ZROWS):
            pltpu.sync_copy(hrows3.at[0, pl.ds(0, ZROWS), :],
                            acc_sh.at[pl.ds(s * WROWS + k * ZROWS, ZROWS), :])

    plsc.subcore_barrier()

    def _prefetch(b):
        j = jbase + b
        p2 = lax.bitwise_and(b, 1)
        p3 = lax.rem(b, 3)
        pltpu.sync_copy(esrc.at[j, 0, :], sidx2.at[p2])
        pltpu.sync_copy(edst.at[j, 0, :], didx3.at[p3])

        @pl.when(c == 0)
        def _():
            pltpu.async_copy(h01.at[sidx2.at[p2]], hrows3.at[p3],
                             semg.at[p2])
            pltpu.async_copy(a_all.at[j, 0, :], attnb2.at[p2, 0],
                             semg.at[p2])
            pltpu.async_copy(a_all.at[j, 1, :], attnb2.at[p2, 1],
                             semg.at[p2])
            pltpu.async_copy(d0.at[didx3.at[p3]], dnb2.at[p2, 0],
                             semg.at[p2])
            pltpu.async_copy(d1.at[didx3.at[p3]], dnb2.at[p2, 1],
                             semg.at[p2])

        @pl.when(c == 1)
        def _():
            pltpu.async_copy(h23.at[sidx2.at[p2]], hrows3.at[p3],
                             semg.at[p2])
            pltpu.async_copy(a_all.at[j, 2, :], attnb2.at[p2, 0],
                             semg.at[p2])
            pltpu.async_copy(a_all.at[j, 3, :], attnb2.at[p2, 1],
                             semg.at[p2])
            pltpu.async_copy(d2.at[didx3.at[p3]], dnb2.at[p2, 0],
                             semg.at[p2])
            pltpu.async_copy(d3.at[didx3.at[p3]], dnb2.at[p2, 1],
                             semg.at[p2])

    _prefetch(0)

    def _blk(b, _):
        p2 = lax.bitwise_and(b, 1)
        p3 = lax.rem(b, 3)
        n3 = lax.rem(b + 1, 3)

        @pl.when(b >= 2)
        def _():
            pltpu.make_async_copy(h01.at[pl.ds(0, BLK3), :],
                                  hrows3.at[n3], sems.at[n3]).wait()

        @pl.when(b + 1 < nb)
        def _():
            _prefetch(b + 1)

        pltpu.make_async_copy(h01.at[pl.ds(0, BLK3), :],
                              hrows3.at[p3], semg.at[p2]).wait()
        pltpu.make_async_copy(a_all.at[0, 0, :], attnb2.at[p2, 0],
                              semg.at[p2]).wait()
        pltpu.make_async_copy(a_all.at[0, 0, :], attnb2.at[p2, 1],
                              semg.at[p2]).wait()
        pltpu.make_async_copy(d0.at[pl.ds(0, BLK3)], dnb2.at[p2, 0],
                              semg.at[p2]).wait()
        pltpu.make_async_copy(d0.at[pl.ds(0, BLK3)], dnb2.at[p2, 1],
                              semg.at[p2]).wait()

        def _mul(g, _):
            av0 = (attnb2[p2, 0, pl.ds(g * 16, 16)]
                   / dnb2[p2, 0, pl.ds(g * 16, 16)])
            av1 = (attnb2[p2, 1, pl.ds(g * 16, 16)]
                   / dnb2[p2, 1, pl.ds(g * 16, 16)])
            for k in range(16):
                e = g * 16 + k
                sp0 = jnp.full((16,), av0[k], jnp.float32)
                sp1 = jnp.full((16,), av1[k], jnp.float32)
                for u in range(OUT_DIM // 16):
                    c0 = u * 16
                    c1 = OUT_DIM + u * 16
                    hrows3[p3, e, pl.ds(c0, 16)] = (
                        hrows3[p3, e, pl.ds(c0, 16)] * sp0)
                    hrows3[p3, e, pl.ds(c1, 16)] = (
                        hrows3[p3, e, pl.ds(c1, 16)] * sp1)
            return 0
        lax.fori_loop(0, BLK3 // 16, _mul, 0)

        pltpu.async_copy(hrows3.at[p3], acc_sh.at[didx3.at[p3]],
                         sems.at[p3], add=True)
        return 0
    lax.fori_loop(0, nb, _blk, 0)

    for tail in range(2):
        lp = lax.rem(nb - 2 + tail, 3)
        pltpu.make_async_copy(h01.at[pl.ds(0, BLK3), :],
                              hrows3.at[lp], sems.at[lp]).wait()

    plsc.subcore_barrier()

    @pl.when(s < WTILES)
    def _():
        for k in range(WROWS // ZROWS):
            r0 = s * WROWS + k * ZROWS
            pltpu.sync_copy(acc_sh.at[pl.ds(r0, ZROWS), :],
                            out_hbm.at[c, pl.ds(r0, ZROWS), :])


def _k3(h01, h23, a_all, dns, esrc, edst):
    mesh = plsc.VectorSubcoreMesh(core_axis_name="c", subcore_axis_name="s",
                                  num_cores=NC, num_subcores=NS)
    f = pl.kernel(
        _k3_body,
        out_type=jax.ShapeDtypeStruct((NC, N, HALF), jnp.float32),
        mesh=mesh,
        scratch_types=[
            pltpu.VMEM((2, BLK3), jnp.int32),
            pltpu.VMEM((3, BLK3), jnp.int32),
            pltpu.VMEM((3, BLK3, HALF), jnp.float32),
            pltpu.VMEM((2, 2, BLK3), jnp.float32),
            pltpu.VMEM((2, 2, BLK3), jnp.float32),
            pltpu.VMEM_SHARED((N, HALF), jnp.float32),
            pltpu.SemaphoreType.DMA((2,)),
            pltpu.SemaphoreType.DMA((3,)),
        ],
    )
    return f(h01, h23, a_all,
             dns[0], dns[1], dns[2], dns[3], esrc, edst)


def _k4_body(acc_ref, g_ref, b_ref, out_ref, sums_ref):
    ph = pl.program_id(0)
    i = pl.program_id(1)
    r = jnp.concatenate([acc_ref[0], acc_ref[1]], axis=-1)
    r = jnp.maximum(r, 0.0)

    @pl.when(ph == 0)
    def _():
        s1 = jnp.sum(r, axis=0, keepdims=True)
        s2 = jnp.sum(r * r, axis=0, keepdims=True)
        cur = jnp.concatenate([s1, s2], axis=0)

        @pl.when(i == 0)
        def _():
            sums_ref[...] = cur

        @pl.when(i > 0)
        def _():
            sums_ref[...] = sums_ref[...] + cur

    @pl.when(ph == 1)
    def _():
        mean = sums_ref[0:1, :] * (1.0 / N)
        var = sums_ref[1:2, :] * (1.0 / N) - mean * mean
        inv = lax.rsqrt(var + 1e-05)
        out_ref[...] = (r - mean) * inv * g_ref[...] + b_ref[...]


def _k4(acc, gamma2, beta2):
    return pl.pallas_call(
        _k4_body,
        grid=(2, GRID),
        in_specs=[
            pl.BlockSpec((NC, MB, HALF), lambda ph, i: (0, i, 0)),
            pl.BlockSpec((1, HD), lambda ph, i: (0, 0)),
            pl.BlockSpec((1, HD), lambda ph, i: (0, 0)),
        ],
        out_specs=pl.BlockSpec((MB, HD), lambda ph, i: (i, 0)),
        out_shape=jax.ShapeDtypeStruct((N, HD), jnp.float32),
        scratch_shapes=[pltpu.VMEM((2, HD), jnp.float32)],
    )(acc, gamma2, beta2)


@jax.jit
def kernel(x, edge_index, W_w, W_b, a_src_w, a_dst_w, bn_gamma, bn_beta):
    wt = W_w.T
    b2 = W_b.reshape(1, HD)
    asrc2 = jnp.tile(a_src_w[0], HEADS).reshape(1, HD)
    adst2 = jnp.tile(a_dst_w[0], HEADS).reshape(1, HD)
    esrc = edge_index[0].reshape(NBLK2, 1, BLK2)
    edst = edge_index[1].reshape(NBLK2, 1, BLK2)

    h01, h23, sst, sdt = _k1(x, wt, b2, asrc2, adst2)
    sstT = sst.T
    sdtT = sdt.T
    ss = [sstT[i] for i in range(HEADS)]
    sd = [sdtT[i] for i in range(HEADS)]
    a_all, dparts = _k2(ss, sd, esrc, edst)
    dn = _k2b(dparts)
    dns = [dn[i] for i in range(HEADS)]
    acc = _k3(h01, h23, a_all, dns, esrc, edst)
    out = _k4(acc, bn_gamma.reshape(1, HD), bn_beta.reshape(1, HD))
    return out
